# baseline (speedup 1.0000x reference)
import functools

import jax
import jax.numpy as jnp
from jax import lax
from jax.experimental import pallas as pl
from jax.experimental.pallas import tpu as pltpu
from jax.experimental.pallas import tpu_sc as plsc

_FIELDS = 26
_FIELD_DIM = 40000
_BATCH = 16384
_NC, _NS, _L = 2, 16, 16
_NW = _NC * _NS
_ROWS_W = _BATCH // _NW
_LOOK_W = _ROWS_W * _FIELDS
_NP = 8
_ROWS_P = _ROWS_W // _NP
_LOOK_P = _ROWS_P * _FIELDS


def _body(x_hbm, table_hbm, bias_hbm, out_hbm, xv, idxv, vals, outv, biasv,
          zidx, semb, sem0, sem1, sem2, sem3, sem4, sem5, sem6, sem7):
    wid = lax.axis_index("s") * _NC + lax.axis_index("c")
    r0 = wid * _ROWS_W
    sems = (sem0, sem1, sem2, sem3, sem4, sem5, sem6, sem7)

    iota = lax.iota(jnp.int32, _L)
    zero16 = jnp.zeros((_L,), jnp.int32)

    zidx[pl.ds(0, _L)] = zero16
    bias_cp = pltpu.async_copy(bias_hbm.at[zidx], biasv, semb)

    x_cps = []
    for q in range(_NP):
        x_cps.append(pltpu.async_copy(
            x_hbm.at[pl.ds(r0 + q * _ROWS_P, _ROWS_P), :],
            xv.at[pl.ds(q * _ROWS_P, _ROWS_P), :],
            sems[q],
        ))

    def make_idx_body(q):
        def idx_body(c, carry):
            rows = q * _ROWS_P + c * _L + iota
            xcols = [
                plsc.load_gather(xv, [rows, jnp.full((_L,), f, jnp.int32)])
                for f in range(_FIELDS)
            ]
            for f in range(_FIELDS):
                o = q * _LOOK_P + f * _ROWS_P + c * _L
                idxv[pl.ds(o, _L)] = xcols[f] + f * _FIELD_DIM
            return carry
        return idx_body

    for q in range(_NP):
        x_cps[q].wait()
        lax.fori_loop(0, _ROWS_P // _L, make_idx_body(q), 0, unroll=4)
        pltpu.async_copy(
            table_hbm.at[idxv.at[pl.ds(q * _LOOK_P, _LOOK_P)]],
            vals.at[pl.ds(q * _LOOK_P, _LOOK_P)],
            sems[q],
        )

    bias_cp.wait()
    bvec = biasv[pl.ds(0, _L)]

    def make_red_body(q):
        def red_body(c, carry):
            b0 = c * _L
            vs = [vals[pl.ds(q * _LOOK_P + f * _ROWS_P + b0, _L)]
                  for f in range(_FIELDS)]
            vs.append(bvec)
            while len(vs) > 1:
                vs = [vs[i] + vs[i + 1] for i in range(0, len(vs) - 1, 2)] + (
                    [vs[-1]] if len(vs) % 2 else [])
            outv[pl.ds(q * _ROWS_P + b0, _L)] = vs[0]
            return carry
        return red_body

    for q in range(_NP):
        pltpu.make_async_copy(
            table_hbm.at[idxv.at[pl.ds(q * _LOOK_P, _LOOK_P)]],
            vals.at[pl.ds(q * _LOOK_P, _LOOK_P)],
            sems[q],
        ).wait()
        lax.fori_loop(0, _ROWS_P // _L, make_red_body(q), 0, unroll=4)

    pltpu.sync_copy(outv, out_hbm.at[pl.ds(r0, _ROWS_W)])


_fl_kernel = functools.partial(
    pl.kernel,
    out_type=jax.ShapeDtypeStruct((_BATCH,), jnp.float32),
    mesh=plsc.VectorSubcoreMesh(
        core_axis_name="c", subcore_axis_name="s",
        num_cores=_NC, num_subcores=_NS,
    ),
    scratch_types=[
        pltpu.VMEM((_ROWS_W, _FIELDS), jnp.int32),
        pltpu.VMEM((_LOOK_W,), jnp.int32),
        pltpu.VMEM((_LOOK_W,), jnp.float32),
        pltpu.VMEM((_ROWS_W,), jnp.float32),
        pltpu.VMEM((_L,), jnp.float32),
        pltpu.VMEM((_L,), jnp.int32),
        pltpu.SemaphoreType.DMA,
        pltpu.SemaphoreType.DMA,
        pltpu.SemaphoreType.DMA,
        pltpu.SemaphoreType.DMA,
        pltpu.SemaphoreType.DMA,
        pltpu.SemaphoreType.DMA,
        pltpu.SemaphoreType.DMA,
        pltpu.SemaphoreType.DMA,
        pltpu.SemaphoreType.DMA,
    ],
    compiler_params=pltpu.CompilerParams(needs_layout_passes=False),
)(_body)


def kernel(x, table, bias):
    out = _fl_kernel(x, table.reshape(-1), bias)
    return out.reshape(_BATCH, 1)


# speedup vs baseline: 1.0193x; 1.0193x over previous
import functools

import jax
import jax.numpy as jnp
from jax import lax
from jax.experimental import pallas as pl
from jax.experimental.pallas import tpu as pltpu
from jax.experimental.pallas import tpu_sc as plsc

_FIELDS = 26
_FIELD_DIM = 40000
_BATCH = 16384
_NC, _NS, _L = 2, 16, 16
_NW = _NC * _NS
_ROWS_W = _BATCH // _NW
_LOOK_W = _ROWS_W * _FIELDS
_NP = 8
_ROWS_P = _ROWS_W // _NP
_LOOK_P = _ROWS_P * _FIELDS


def _body(x_hbm, table_hbm, bias_hbm, out_hbm, xv, idxv, vals, outv, biasv,
          zidx, semb, sem0, sem1, sem2, sem3, sem4, sem5, sem6, sem7):
    wid = lax.axis_index("s") * _NC + lax.axis_index("c")
    r0 = wid * _ROWS_W
    sems = (sem0, sem1, sem2, sem3, sem4, sem5, sem6, sem7)

    iota = lax.iota(jnp.int32, _L)
    zero16 = jnp.zeros((_L,), jnp.int32)

    zidx[pl.ds(0, _L)] = zero16
    bias_cp = pltpu.async_copy(bias_hbm.at[zidx], biasv, semb)

    x_cps = []
    for q in range(_NP):
        x_cps.append(pltpu.async_copy(
            x_hbm.at[pl.ds(r0 + q * _ROWS_P, _ROWS_P), :],
            xv.at[pl.ds(q * _ROWS_P, _ROWS_P), :],
            sems[q],
        ))

    def make_idx_body(q):
        def idx_body(c, carry):
            rows = q * _ROWS_P + c * _L + iota
            xcols = [
                plsc.load_gather(xv, [rows, jnp.full((_L,), f, jnp.int32)])
                for f in range(_FIELDS)
            ]
            for f in range(_FIELDS):
                o = q * _LOOK_P + f * _ROWS_P + c * _L
                idxv[pl.ds(o, _L)] = xcols[f] + f * _FIELD_DIM
            return carry
        return idx_body

    for q in range(_NP):
        x_cps[q].wait()
        lax.fori_loop(0, _ROWS_P // _L, make_idx_body(q), 0, unroll=2)
        pltpu.async_copy(
            table_hbm.at[idxv.at[pl.ds(q * _LOOK_P, _LOOK_P)]],
            vals.at[pl.ds(q * _LOOK_P, _LOOK_P)],
            sems[q],
        )

    bias_cp.wait()
    bvec = biasv[pl.ds(0, _L)]

    def make_red_body(q):
        def red_body(c, carry):
            b0 = c * _L
            vs = [vals[pl.ds(q * _LOOK_P + f * _ROWS_P + b0, _L)]
                  for f in range(_FIELDS)]
            vs.append(bvec)
            while len(vs) > 1:
                vs = [vs[i] + vs[i + 1] for i in range(0, len(vs) - 1, 2)] + (
                    [vs[-1]] if len(vs) % 2 else [])
            outv[pl.ds(q * _ROWS_P + b0, _L)] = vs[0]
            return carry
        return red_body

    for q in range(_NP):
        pltpu.make_async_copy(
            table_hbm.at[idxv.at[pl.ds(q * _LOOK_P, _LOOK_P)]],
            vals.at[pl.ds(q * _LOOK_P, _LOOK_P)],
            sems[q],
        ).wait()
        lax.fori_loop(0, _ROWS_P // _L, make_red_body(q), 0, unroll=2)

    pltpu.sync_copy(outv, out_hbm.at[pl.ds(r0, _ROWS_W)])


_fl_kernel = functools.partial(
    pl.kernel,
    out_type=jax.ShapeDtypeStruct((_BATCH,), jnp.float32),
    mesh=plsc.VectorSubcoreMesh(
        core_axis_name="c", subcore_axis_name="s",
        num_cores=_NC, num_subcores=_NS,
    ),
    scratch_types=[
        pltpu.VMEM((_ROWS_W, _FIELDS), jnp.int32),
        pltpu.VMEM((_LOOK_W,), jnp.int32),
        pltpu.VMEM((_LOOK_W,), jnp.float32),
        pltpu.VMEM((_ROWS_W,), jnp.float32),
        pltpu.VMEM((_L,), jnp.float32),
        pltpu.VMEM((_L,), jnp.int32),
        pltpu.SemaphoreType.DMA,
        pltpu.SemaphoreType.DMA,
        pltpu.SemaphoreType.DMA,
        pltpu.SemaphoreType.DMA,
        pltpu.SemaphoreType.DMA,
        pltpu.SemaphoreType.DMA,
        pltpu.SemaphoreType.DMA,
        pltpu.SemaphoreType.DMA,
        pltpu.SemaphoreType.DMA,
    ],
    compiler_params=pltpu.CompilerParams(needs_layout_passes=False),
)(_body)


def kernel(x, table, bias):
    out = _fl_kernel(x, table.reshape(-1), bias)
    return out.reshape(_BATCH, 1)
